# baseline (speedup 1.0000x reference)
import functools

import jax
import jax.numpy as jnp
from jax import lax
from jax.experimental import pallas as pl
from jax.experimental.pallas import tpu as pltpu
from jax.experimental.pallas import tpu_sc as plsc

_N = 2048
_E = 32768
_HID = 16
_NT = 16
_RPT = _N // _NT
_EPT = _E // _NT
_CHUNK = 128
_NCH = _EPT // _CHUNK


def _mm1_body(x_ref, w_ref, o_ref):
    o_ref[...] = jnp.dot(x_ref[...], w_ref[...],
                         preferred_element_type=jnp.float32)


def _mm2_body(g_ref, w_ref, b_ref, o_ref):
    o_ref[...] = jnp.dot(g_ref[...], w_ref[...],
                         preferred_element_type=jnp.float32) + b_ref[...]


_MB = 256

_mm1 = pl.pallas_call(
    _mm1_body,
    grid=(_N // _MB,),
    in_specs=[
        pl.BlockSpec((_MB, _N), lambda i: (i, 0)),
        pl.BlockSpec((_N, _HID), lambda i: (0, 0)),
    ],
    out_specs=pl.BlockSpec((_MB, _HID), lambda i: (i, 0)),
    out_shape=jax.ShapeDtypeStruct((_N, _HID), jnp.float32),
)

_mm2 = pl.pallas_call(
    _mm2_body,
    grid=(_N // _MB,),
    in_specs=[
        pl.BlockSpec((_MB, _HID), lambda i: (i, 0)),
        pl.BlockSpec((_HID, _N), lambda i: (0, 0)),
        pl.BlockSpec((1, _N), lambda i: (0, 0)),
    ],
    out_specs=pl.BlockSpec((_MB, _N), lambda i: (i, 0)),
    out_shape=jax.ShapeDtypeStruct((_N, _N), jnp.float32),
)


def _sc_body(ei_hbm, hpre_hbm, b1_hbm, out_hbm, *refs):
    sidx = refs[0:_NCH]
    didx = refs[_NCH:2 * _NCH]
    (rowbig, hbuf, disbuf, onesbuf, accbuf, b1buf,
     s_deg, s_h, s_acc, s_acc2, sem_i, sem_g, sem_s) = refs[2 * _NCH:]
    cid = lax.axis_index("c")
    wid = lax.axis_index("s")

    @pl.when(cid == 0)
    def _core0_work():
        rows = pl.ds(wid * _RPT, _RPT)
        ebase = wid * _EPT

        hh = [pltpu.async_copy(
            ei_hbm.at[0, pl.ds(ebase + j * _CHUNK, _CHUNK)], sidx[j], sem_i)
            for j in range(_NCH)]
        hh.append(pltpu.async_copy(hpre_hbm.at[rows], hbuf, sem_i))
        hh.append(pltpu.async_copy(b1_hbm, b1buf, sem_i))

        ones = jnp.ones((_HID,), jnp.float32)

        def fill_ones(i, c):
            onesbuf[i, :] = ones
            return c

        lax.fori_loop(0, _RPT, fill_ones, 0)
        for h in hh:
            h.wait()
        hh = [pltpu.async_copy(
            ei_hbm.at[1, pl.ds(ebase + j * _CHUNK, _CHUNK)], didx[j], sem_i)
            for j in range(_NCH)]
        pltpu.sync_copy(onesbuf, s_deg.at[rows])
        for h in hh:
            h.wait()
        plsc.subcore_barrier()

        hh = [pltpu.async_copy(onesbuf, s_deg.at[didx[j]], sem_s, add=True)
              for j in range(_NCH)]
        for h in hh:
            h.wait()
        plsc.subcore_barrier()

        pltpu.sync_copy(s_deg.at[rows], disbuf)

        def mk_dis(i, c):
            d = disbuf[i, :]
            bits = lax.bitcast_convert_type(d, jnp.int32)
            bits = jnp.int32(0x5F3759DF) - lax.shift_right_logical(bits, 1)
            y = lax.bitcast_convert_type(bits, jnp.float32)
            half = 0.5 * d
            y = y * (1.5 - half * y * y)
            y = y * (1.5 - half * y * y)
            y = y * (1.5 - half * y * y)
            disbuf[i, :] = y
            hbuf[i, :] = hbuf[i, :] * y
            return c

        lax.fori_loop(0, _RPT, mk_dis, 0)
        pltpu.sync_copy(hbuf, s_h.at[rows])
        pltpu.sync_copy(hbuf, s_acc.at[rows])
        plsc.subcore_barrier()

        def prop_round(s_to):
            gh = [pltpu.async_copy(
                s_h.at[sidx[j]],
                rowbig.at[pl.ds(j * _CHUNK, _CHUNK)], sem_g)
                for j in range(_NCH)]
            sh = []
            for j in range(_NCH):
                gh[j].wait()
                sh.append(pltpu.async_copy(
                    rowbig.at[pl.ds(j * _CHUNK, _CHUNK)],
                    s_to.at[didx[j]], sem_s, add=True))
            for h in sh:
                h.wait()

        prop_round(s_acc)
        plsc.subcore_barrier()

        pltpu.sync_copy(s_acc.at[rows], accbuf)
        b1v = b1buf[:]

        def mk_h1(i, c):
            a = accbuf[i, :] * disbuf[i, :] + b1v
            a = jnp.maximum(a, 0.0)
            accbuf[i, :] = a * disbuf[i, :]
            return c

        lax.fori_loop(0, _RPT, mk_h1, 0)
        pltpu.sync_copy(accbuf, s_h.at[rows])
        pltpu.sync_copy(accbuf, s_acc2.at[rows])
        plsc.subcore_barrier()

        prop_round(s_acc2)
        plsc.subcore_barrier()

        pltpu.sync_copy(s_acc2.at[rows], accbuf)

        def mk_out(i, c):
            accbuf[i, :] = accbuf[i, :] * disbuf[i, :]
            return c

        lax.fori_loop(0, _RPT, mk_out, 0)
        pltpu.sync_copy(accbuf, out_hbm.at[rows])


_sc_prop = functools.partial(
    pl.kernel,
    mesh=plsc.VectorSubcoreMesh(core_axis_name="c", subcore_axis_name="s"),
    compiler_params=pltpu.CompilerParams(use_tc_tiling_on_sc=False),
    out_type=jax.ShapeDtypeStruct((_N, _HID), jnp.float32),
    scratch_types=(
        [pltpu.VMEM((_CHUNK,), jnp.int32) for _ in range(2 * _NCH)] + [
            pltpu.VMEM((_EPT, _HID), jnp.float32),
            pltpu.VMEM((_RPT, _HID), jnp.float32),
            pltpu.VMEM((_RPT, _HID), jnp.float32),
            pltpu.VMEM((_RPT, _HID), jnp.float32),
            pltpu.VMEM((_RPT, _HID), jnp.float32),
            pltpu.VMEM((_HID,), jnp.float32),
            pltpu.VMEM_SHARED((_N, _HID), jnp.float32),
            pltpu.VMEM_SHARED((_N, _HID), jnp.float32),
            pltpu.VMEM_SHARED((_N, _HID), jnp.float32),
            pltpu.VMEM_SHARED((_N, _HID), jnp.float32),
            pltpu.SemaphoreType.DMA,
            pltpu.SemaphoreType.DMA,
            pltpu.SemaphoreType.DMA,
        ]),
)(_sc_body)


@jax.jit
def kernel(x, edge_index, W1, b1, W2, b2):
    hpre = _mm1(x, W1)
    g = _sc_prop(edge_index, hpre, b1)
    out = _mm2(g, W2, b2.reshape(1, _N))
    return out


# speedup vs baseline: 1.0608x; 1.0608x over previous
import functools

import jax
import jax.numpy as jnp
from jax import lax
from jax.experimental import pallas as pl
from jax.experimental.pallas import tpu as pltpu
from jax.experimental.pallas import tpu_sc as plsc

_N = 2048
_E = 32768
_HID = 16
_NT = 16
_RPT = _N // _NT
_EPT = _E // _NT
_CHUNK = 128
_NCH = _EPT // _CHUNK
_MB = 512


def _mm1_body(x_ref, w_ref, o_ref):
    o_ref[...] = jnp.dot(x_ref[...], w_ref[...],
                         preferred_element_type=jnp.float32)


def _mm2_body(g_ref, w_ref, b_ref, o_ref):
    o_ref[...] = jnp.dot(g_ref[...], w_ref[...],
                         preferred_element_type=jnp.float32) + b_ref[...]


_mm1 = pl.pallas_call(
    _mm1_body,
    out_shape=jax.ShapeDtypeStruct((_N, _HID), jnp.float32),
)

_mm2 = pl.pallas_call(
    _mm2_body,
    out_shape=jax.ShapeDtypeStruct((_N, _N), jnp.float32),
)


def _sc_body(ei_hbm, hpre_hbm, b1_hbm, out_hbm, *refs):
    sidx = refs[0:_NCH]
    didx = refs[_NCH:2 * _NCH]
    (rowbig, hbuf, disbuf, onesbuf, accbuf, b1buf,
     s_deg, s_h, s_acc, s_acc2, sem_i, sem_g, sem_s) = refs[2 * _NCH:]
    cid = lax.axis_index("c")
    wid = lax.axis_index("s")

    @pl.when(cid == 0)
    def _core0_work():
        rows = pl.ds(wid * _RPT, _RPT)
        ebase = wid * _EPT

        hh = [pltpu.async_copy(
            ei_hbm.at[pl.ds(ebase + j * _CHUNK, _CHUNK)], sidx[j], sem_i)
            for j in range(_NCH)]
        hh.append(pltpu.async_copy(hpre_hbm.at[rows], hbuf, sem_i))
        hh.append(pltpu.async_copy(b1_hbm, b1buf, sem_i))

        ones = jnp.ones((_HID,), jnp.float32)

        def fill_ones(i, c):
            onesbuf[i, :] = ones
            return c

        lax.fori_loop(0, _RPT, fill_ones, 0)
        for h in hh:
            h.wait()
        hh = [pltpu.async_copy(
            ei_hbm.at[pl.ds(_E + ebase + j * _CHUNK, _CHUNK)], didx[j], sem_i)
            for j in range(_NCH)]
        pltpu.sync_copy(onesbuf, s_deg.at[rows])
        for h in hh:
            h.wait()
        plsc.subcore_barrier()

        hh = [pltpu.async_copy(onesbuf, s_deg.at[didx[j]], sem_s, add=True)
              for j in range(_NCH)]
        for h in hh:
            h.wait()
        plsc.subcore_barrier()

        pltpu.sync_copy(s_deg.at[rows], disbuf)

        def mk_dis(i, c):
            for i2 in (2 * i, 2 * i + 1):
                d = disbuf[i2, :]
                bits = lax.bitcast_convert_type(d, jnp.int32)
                bits = (jnp.int32(0x5F3759DF)
                        - lax.shift_right_logical(bits, 1))
                y = lax.bitcast_convert_type(bits, jnp.float32)
                half = 0.5 * d
                y = y * (1.5 - half * y * y)
                y = y * (1.5 - half * y * y)
                y = y * (1.5 - half * y * y)
                disbuf[i2, :] = y
                accbuf[i2, :] = hbuf[i2, :] * y
            return c

        lax.fori_loop(0, _RPT // 2, mk_dis, 0)
        pltpu.sync_copy(accbuf, s_h.at[rows])
        pltpu.sync_copy(accbuf, s_acc.at[rows])
        plsc.subcore_barrier()

        def prop_round(s_to):
            gh = [pltpu.async_copy(
                s_h.at[sidx[j]],
                rowbig.at[pl.ds(j * _CHUNK, _CHUNK)], sem_g)
                for j in range(_NCH)]
            sh = []
            for j in range(_NCH):
                gh[j].wait()
                sh.append(pltpu.async_copy(
                    rowbig.at[pl.ds(j * _CHUNK, _CHUNK)],
                    s_to.at[didx[j]], sem_s, add=True))
            for h in sh:
                h.wait()

        prop_round(s_acc)
        plsc.subcore_barrier()

        pltpu.sync_copy(s_acc.at[rows], accbuf)
        b1v = b1buf[:]

        def mk_h1(i, c):
            for i2 in (2 * i, 2 * i + 1):
                a = accbuf[i2, :] * disbuf[i2, :] + b1v
                a = jnp.maximum(a, 0.0)
                accbuf[i2, :] = a * disbuf[i2, :]
            return c

        lax.fori_loop(0, _RPT // 2, mk_h1, 0)
        pltpu.sync_copy(accbuf, s_h.at[rows])
        pltpu.sync_copy(accbuf, s_acc2.at[rows])
        plsc.subcore_barrier()

        prop_round(s_acc2)
        plsc.subcore_barrier()

        pltpu.sync_copy(s_acc2.at[rows], accbuf)

        def mk_out(i, c):
            for i2 in (2 * i, 2 * i + 1):
                accbuf[i2, :] = accbuf[i2, :] * disbuf[i2, :]
            return c

        lax.fori_loop(0, _RPT // 2, mk_out, 0)
        pltpu.sync_copy(accbuf, out_hbm.at[rows])


_sc_prop = functools.partial(
    pl.kernel,
    mesh=plsc.VectorSubcoreMesh(core_axis_name="c", subcore_axis_name="s"),
    compiler_params=pltpu.CompilerParams(use_tc_tiling_on_sc=False),
    out_type=jax.ShapeDtypeStruct((_N, _HID), jnp.float32),
    scratch_types=(
        [pltpu.VMEM((_CHUNK,), jnp.int32) for _ in range(2 * _NCH)] + [
            pltpu.VMEM((_EPT, _HID), jnp.float32),
            pltpu.VMEM((_RPT, _HID), jnp.float32),
            pltpu.VMEM((_RPT, _HID), jnp.float32),
            pltpu.VMEM((_RPT, _HID), jnp.float32),
            pltpu.VMEM((_RPT, _HID), jnp.float32),
            pltpu.VMEM((_HID,), jnp.float32),
            pltpu.VMEM_SHARED((_N, _HID), jnp.float32),
            pltpu.VMEM_SHARED((_N, _HID), jnp.float32),
            pltpu.VMEM_SHARED((_N, _HID), jnp.float32),
            pltpu.VMEM_SHARED((_N, _HID), jnp.float32),
            pltpu.SemaphoreType.DMA,
            pltpu.SemaphoreType.DMA,
            pltpu.SemaphoreType.DMA,
        ]),
)(_sc_body)


@jax.jit
def kernel(x, edge_index, W1, b1, W2, b2):
    ei_flat = edge_index.reshape(2 * _E)
    hpre = _mm1(x, W1)
    g = _sc_prop(ei_flat, hpre, b1)
    out = _mm2(g, W2, b2.reshape(1, _N))
    return out


# speedup vs baseline: 1.1012x; 1.0381x over previous
import functools

import jax
import jax.numpy as jnp
from jax import lax
from jax.experimental import pallas as pl
from jax.experimental.pallas import tpu as pltpu
from jax.experimental.pallas import tpu_sc as plsc

_N = 2048
_E = 32768
_HID = 16
_NT = 16
_RPT = _N // _NT
_EPT = _E // _NT
_CHUNK = 128
_NCH = _EPT // _CHUNK


def _mm1_body(x_ref, w_ref, o_ref):
    o_ref[...] = jnp.dot(x_ref[...], w_ref[...],
                         preferred_element_type=jnp.float32)


def _mm2_body(g_ref, w_ref, b_ref, o_ref):
    o_ref[...] = jnp.dot(g_ref[...], w_ref[...],
                         preferred_element_type=jnp.float32) + b_ref[...]


_mm1 = pl.pallas_call(
    _mm1_body,
    out_shape=jax.ShapeDtypeStruct((_N, _HID), jnp.float32),
)

_mm2 = pl.pallas_call(
    _mm2_body,
    out_shape=jax.ShapeDtypeStruct((_N, _N), jnp.float32),
)


def _sc_deg(ei_hbm, dis_hbm, *refs):
    didx = refs[0:_NCH]
    (disbuf, onesbuf, s_deg, sem_i, sem_s) = refs[_NCH:]
    cid = lax.axis_index("c")
    wid = lax.axis_index("s")

    @pl.when(cid == 0)
    def _core0_work():
        rows = pl.ds(wid * _RPT, _RPT)
        ebase = wid * _EPT

        hh = [pltpu.async_copy(
            ei_hbm.at[pl.ds(_E + ebase + j * _CHUNK, _CHUNK)], didx[j],
            sem_i) for j in range(_NCH)]

        ones = jnp.ones((_HID,), jnp.float32)

        def fill_ones(i, c):
            onesbuf[i, :] = ones
            return c

        lax.fori_loop(0, _RPT, fill_ones, 0)
        pltpu.sync_copy(onesbuf, s_deg.at[rows])
        for h in hh:
            h.wait()
        plsc.subcore_barrier()

        hh = [pltpu.async_copy(onesbuf, s_deg.at[didx[j]], sem_s, add=True)
              for j in range(_NCH)]
        for h in hh:
            h.wait()
        plsc.subcore_barrier()

        pltpu.sync_copy(s_deg.at[rows], disbuf)

        def mk_dis(i, c):
            for i2 in (2 * i, 2 * i + 1):
                d = disbuf[i2, :]
                bits = lax.bitcast_convert_type(d, jnp.int32)
                bits = (jnp.int32(0x5F3759DF)
                        - lax.shift_right_logical(bits, 1))
                y = lax.bitcast_convert_type(bits, jnp.float32)
                half = 0.5 * d
                y = y * (1.5 - half * y * y)
                y = y * (1.5 - half * y * y)
                y = y * (1.5 - half * y * y)
                disbuf[i2, :] = y
            return c

        lax.fori_loop(0, _RPT // 2, mk_dis, 0)
        pltpu.sync_copy(disbuf, dis_hbm.at[rows])


def _sc_body(ei_hbm, hpre_hbm, dis_hbm, b1_hbm, out_hbm, *refs):
    sidx = refs[0:_NCH]
    didx = refs[_NCH:2 * _NCH]
    (rowbig, hbuf, disbuf, accbuf, b1buf,
     s_h, s_acc, s_acc2, sem_i, sem_g, sem_s) = refs[2 * _NCH:]
    cid = lax.axis_index("c")
    wid = lax.axis_index("s")

    @pl.when(cid == 0)
    def _core0_work():
        rows = pl.ds(wid * _RPT, _RPT)
        ebase = wid * _EPT

        hh = [pltpu.async_copy(
            ei_hbm.at[pl.ds(ebase + j * _CHUNK, _CHUNK)], sidx[j], sem_i)
            for j in range(_NCH)]
        hh.append(pltpu.async_copy(hpre_hbm.at[rows], hbuf, sem_i))
        for h in hh:
            h.wait()
        hh = [pltpu.async_copy(
            ei_hbm.at[pl.ds(_E + ebase + j * _CHUNK, _CHUNK)], didx[j],
            sem_i) for j in range(_NCH)]
        hh.append(pltpu.async_copy(dis_hbm.at[rows], disbuf, sem_i))
        hh.append(pltpu.async_copy(b1_hbm, b1buf, sem_i))
        for h in hh:
            h.wait()

        def mk_hs(i, c):
            for i2 in (2 * i, 2 * i + 1):
                accbuf[i2, :] = hbuf[i2, :] * disbuf[i2, :]
            return c

        lax.fori_loop(0, _RPT // 2, mk_hs, 0)
        pltpu.sync_copy(accbuf, s_h.at[rows])
        pltpu.sync_copy(accbuf, s_acc.at[rows])
        plsc.subcore_barrier()

        def prop_round(s_to):
            gh = [pltpu.async_copy(
                s_h.at[sidx[j]],
                rowbig.at[pl.ds(j * _CHUNK, _CHUNK)], sem_g)
                for j in range(_NCH)]
            sh = []
            for j in range(_NCH):
                gh[j].wait()
                sh.append(pltpu.async_copy(
                    rowbig.at[pl.ds(j * _CHUNK, _CHUNK)],
                    s_to.at[didx[j]], sem_s, add=True))
            for h in sh:
                h.wait()

        prop_round(s_acc)
        plsc.subcore_barrier()

        pltpu.sync_copy(s_acc.at[rows], accbuf)
        b1v = b1buf[:]

        def mk_h1(i, c):
            for i2 in (2 * i, 2 * i + 1):
                a = accbuf[i2, :] * disbuf[i2, :] + b1v
                a = jnp.maximum(a, 0.0)
                accbuf[i2, :] = a * disbuf[i2, :]
            return c

        lax.fori_loop(0, _RPT // 2, mk_h1, 0)
        pltpu.sync_copy(accbuf, s_h.at[rows])
        pltpu.sync_copy(accbuf, s_acc2.at[rows])
        plsc.subcore_barrier()

        prop_round(s_acc2)
        plsc.subcore_barrier()

        pltpu.sync_copy(s_acc2.at[rows], accbuf)

        def mk_out(i, c):
            for i2 in (2 * i, 2 * i + 1):
                accbuf[i2, :] = accbuf[i2, :] * disbuf[i2, :]
            return c

        lax.fori_loop(0, _RPT // 2, mk_out, 0)
        pltpu.sync_copy(accbuf, out_hbm.at[rows])


_mesh = plsc.VectorSubcoreMesh(core_axis_name="c", subcore_axis_name="s")

_sc_deg_call = functools.partial(
    pl.kernel,
    mesh=_mesh,
    compiler_params=pltpu.CompilerParams(use_tc_tiling_on_sc=False),
    out_type=jax.ShapeDtypeStruct((_N, _HID), jnp.float32),
    scratch_types=(
        [pltpu.VMEM((_CHUNK,), jnp.int32) for _ in range(_NCH)] + [
            pltpu.VMEM((_RPT, _HID), jnp.float32),
            pltpu.VMEM((_RPT, _HID), jnp.float32),
            pltpu.VMEM_SHARED((_N, _HID), jnp.float32),
            pltpu.SemaphoreType.DMA,
            pltpu.SemaphoreType.DMA,
        ]),
)(_sc_deg)

_sc_prop = functools.partial(
    pl.kernel,
    mesh=_mesh,
    compiler_params=pltpu.CompilerParams(use_tc_tiling_on_sc=False),
    out_type=jax.ShapeDtypeStruct((_N, _HID), jnp.float32),
    scratch_types=(
        [pltpu.VMEM((_CHUNK,), jnp.int32) for _ in range(2 * _NCH)] + [
            pltpu.VMEM((_EPT, _HID), jnp.float32),
            pltpu.VMEM((_RPT, _HID), jnp.float32),
            pltpu.VMEM((_RPT, _HID), jnp.float32),
            pltpu.VMEM((_RPT, _HID), jnp.float32),
            pltpu.VMEM((_HID,), jnp.float32),
            pltpu.VMEM_SHARED((_N, _HID), jnp.float32),
            pltpu.VMEM_SHARED((_N, _HID), jnp.float32),
            pltpu.VMEM_SHARED((_N, _HID), jnp.float32),
            pltpu.SemaphoreType.DMA,
            pltpu.SemaphoreType.DMA,
            pltpu.SemaphoreType.DMA,
        ]),
)(_sc_body)


@jax.jit
def kernel(x, edge_index, W1, b1, W2, b2):
    ei_flat = edge_index.reshape(2 * _E)
    dis = _sc_deg_call(ei_flat)
    hpre = _mm1(x, W1)
    g = _sc_prop(ei_flat, hpre, dis, b1)
    out = _mm2(g, W2, b2.reshape(1, _N))
    return out


# speedup vs baseline: 1.1623x; 1.0555x over previous
import functools

import jax
import jax.numpy as jnp
from jax import lax
from jax.experimental import pallas as pl
from jax.experimental.pallas import tpu as pltpu
from jax.experimental.pallas import tpu_sc as plsc

_N = 2048
_E = 32768
_HID = 16
_NT = 16
_RPT = _N // _NT
_EPT = _E // _NT
_CHUNK = 128
_NCH = _EPT // _CHUNK


_LW = 128


def _mm1_body(x_ref, w_ref, o_ref):
    r = jnp.dot(x_ref[...], w_ref[...], preferred_element_type=jnp.float32)
    o_ref[...] = jnp.concatenate(
        [r, jnp.zeros((_N, _LW - _HID), jnp.float32)], axis=1)


def _mm2_body(g_ref, w_ref, b_ref, o_ref):
    w = jnp.concatenate(
        [w_ref[...], jnp.zeros((_LW - _HID, _N), jnp.float32)], axis=0)
    o_ref[...] = jnp.dot(g_ref[...], w,
                         preferred_element_type=jnp.float32) + b_ref[...]


_mm1 = pl.pallas_call(
    _mm1_body,
    out_shape=jax.ShapeDtypeStruct((_N, _LW), jnp.float32),
)

_mm2 = pl.pallas_call(
    _mm2_body,
    out_shape=jax.ShapeDtypeStruct((_N, _N), jnp.float32),
)


def _sc_deg(ei_hbm, dis_hbm, *refs):
    didx = refs[0:_NCH]
    (disbuf, onesbuf, s_deg, sem_i, sem_s) = refs[_NCH:]
    cid = lax.axis_index("c")
    wid = lax.axis_index("s")

    @pl.when(cid == 0)
    def _core0_work():
        rows = pl.ds(wid * _RPT, _RPT)
        ebase = wid * _EPT

        hh = [pltpu.async_copy(
            ei_hbm.at[pl.ds(_E + ebase + j * _CHUNK, _CHUNK)], didx[j],
            sem_i) for j in range(_NCH)]

        ones = jnp.ones((_HID,), jnp.float32)

        def fill_ones(i, c):
            onesbuf[i, :] = ones
            return c

        lax.fori_loop(0, _RPT, fill_ones, 0)
        pltpu.sync_copy(onesbuf, s_deg.at[rows])
        for h in hh:
            h.wait()
        plsc.subcore_barrier()

        hh = [pltpu.async_copy(onesbuf, s_deg.at[didx[j]], sem_s, add=True)
              for j in range(_NCH)]
        for h in hh:
            h.wait()
        plsc.subcore_barrier()

        pltpu.sync_copy(s_deg.at[rows], disbuf)

        def mk_dis(i, c):
            for i2 in (2 * i, 2 * i + 1):
                d = disbuf[i2, :]
                bits = lax.bitcast_convert_type(d, jnp.int32)
                bits = (jnp.int32(0x5F3759DF)
                        - lax.shift_right_logical(bits, 1))
                y = lax.bitcast_convert_type(bits, jnp.float32)
                half = 0.5 * d
                y = y * (1.5 - half * y * y)
                y = y * (1.5 - half * y * y)
                y = y * (1.5 - half * y * y)
                disbuf[i2, :] = y
            return c

        lax.fori_loop(0, _RPT // 2, mk_dis, 0)
        pltpu.sync_copy(disbuf, dis_hbm.at[rows])


def _sc_body(ei_hbm, hpre_hbm, dis_hbm, b1_hbm, out_hbm, *refs):
    sidx = refs[0:_NCH]
    didx = refs[_NCH:2 * _NCH]
    (rowbig, hbuf, disbuf, accbuf, outbuf, b1buf,
     s_h, s_acc, s_acc2, sem_i, sem_g, sem_s) = refs[2 * _NCH:]
    cid = lax.axis_index("c")
    wid = lax.axis_index("s")

    @pl.when(cid == 0)
    def _core0_work():
        rows = pl.ds(wid * _RPT, _RPT)
        ebase = wid * _EPT

        hh = [pltpu.async_copy(
            ei_hbm.at[pl.ds(ebase + j * _CHUNK, _CHUNK)], sidx[j], sem_i)
            for j in range(_NCH)]
        hh.append(pltpu.async_copy(hpre_hbm.at[rows], hbuf, sem_i))
        zv = jnp.zeros((_HID,), jnp.float32)

        def fill_zero(i, c):
            for k in range(_LW // _HID):
                outbuf[i, pl.ds(k * _HID, _HID)] = zv
            return c

        lax.fori_loop(0, _RPT, fill_zero, 0)
        for h in hh:
            h.wait()
        hh = [pltpu.async_copy(
            ei_hbm.at[pl.ds(_E + ebase + j * _CHUNK, _CHUNK)], didx[j],
            sem_i) for j in range(_NCH)]
        hh.append(pltpu.async_copy(dis_hbm.at[rows], disbuf, sem_i))
        hh.append(pltpu.async_copy(b1_hbm, b1buf, sem_i))
        for h in hh:
            h.wait()

        def mk_hs(i, c):
            for i2 in (2 * i, 2 * i + 1):
                accbuf[i2, :] = hbuf[i2, pl.ds(0, _HID)] * disbuf[i2, :]
            return c

        lax.fori_loop(0, _RPT // 2, mk_hs, 0)
        pltpu.sync_copy(accbuf, s_h.at[rows])
        pltpu.sync_copy(accbuf, s_acc.at[rows])
        plsc.subcore_barrier()

        def prop_round(s_to):
            gh = [pltpu.async_copy(
                s_h.at[sidx[j]],
                rowbig.at[pl.ds(j * _CHUNK, _CHUNK)], sem_g)
                for j in range(_NCH)]
            sh = []
            for j in range(_NCH):
                gh[j].wait()
                sh.append(pltpu.async_copy(
                    rowbig.at[pl.ds(j * _CHUNK, _CHUNK)],
                    s_to.at[didx[j]], sem_s, add=True))
            for h in sh:
                h.wait()

        prop_round(s_acc)
        plsc.subcore_barrier()

        pltpu.sync_copy(s_acc.at[rows], accbuf)
        b1v = b1buf[:]

        def mk_h1(i, c):
            for i2 in (2 * i, 2 * i + 1):
                a = accbuf[i2, :] * disbuf[i2, :] + b1v
                a = jnp.maximum(a, 0.0)
                accbuf[i2, :] = a * disbuf[i2, :]
            return c

        lax.fori_loop(0, _RPT // 2, mk_h1, 0)
        pltpu.sync_copy(accbuf, s_h.at[rows])
        pltpu.sync_copy(accbuf, s_acc2.at[rows])
        plsc.subcore_barrier()

        prop_round(s_acc2)
        plsc.subcore_barrier()

        pltpu.sync_copy(s_acc2.at[rows], accbuf)

        def mk_out(i, c):
            for i2 in (2 * i, 2 * i + 1):
                outbuf[i2, pl.ds(0, _HID)] = (
                    accbuf[i2, :] * disbuf[i2, :])
            return c

        lax.fori_loop(0, _RPT // 2, mk_out, 0)
        pltpu.sync_copy(outbuf, out_hbm.at[rows])


_mesh = plsc.VectorSubcoreMesh(core_axis_name="c", subcore_axis_name="s")

_sc_deg_call = functools.partial(
    pl.kernel,
    mesh=_mesh,
    compiler_params=pltpu.CompilerParams(use_tc_tiling_on_sc=False),
    out_type=jax.ShapeDtypeStruct((_N, _HID), jnp.float32),
    scratch_types=(
        [pltpu.VMEM((_CHUNK,), jnp.int32) for _ in range(_NCH)] + [
            pltpu.VMEM((_RPT, _HID), jnp.float32),
            pltpu.VMEM((_RPT, _HID), jnp.float32),
            pltpu.VMEM_SHARED((_N, _HID), jnp.float32),
            pltpu.SemaphoreType.DMA,
            pltpu.SemaphoreType.DMA,
        ]),
)(_sc_deg)

_sc_prop = functools.partial(
    pl.kernel,
    mesh=_mesh,
    compiler_params=pltpu.CompilerParams(use_tc_tiling_on_sc=False),
    out_type=jax.ShapeDtypeStruct((_N, _LW), jnp.float32),
    scratch_types=(
        [pltpu.VMEM((_CHUNK,), jnp.int32) for _ in range(2 * _NCH)] + [
            pltpu.VMEM((_EPT, _HID), jnp.float32),
            pltpu.VMEM((_RPT, _LW), jnp.float32),
            pltpu.VMEM((_RPT, _HID), jnp.float32),
            pltpu.VMEM((_RPT, _HID), jnp.float32),
            pltpu.VMEM((_RPT, _LW), jnp.float32),
            pltpu.VMEM((_HID,), jnp.float32),
            pltpu.VMEM_SHARED((_N, _HID), jnp.float32),
            pltpu.VMEM_SHARED((_N, _HID), jnp.float32),
            pltpu.VMEM_SHARED((_N, _HID), jnp.float32),
            pltpu.SemaphoreType.DMA,
            pltpu.SemaphoreType.DMA,
            pltpu.SemaphoreType.DMA,
        ]),
)(_sc_body)


@jax.jit
def kernel(x, edge_index, W1, b1, W2, b2):
    ei_flat = edge_index.reshape(2 * _E)
    dis = _sc_deg_call(ei_flat)
    hpre = _mm1(x, W1)
    g = _sc_prop(ei_flat, hpre, dis, b1)
    out = _mm2(g, W2, b2.reshape(1, _N))
    return out


# speedup vs baseline: 1.1638x; 1.0013x over previous
import functools

import jax
import jax.numpy as jnp
from jax import lax
from jax.experimental import pallas as pl
from jax.experimental.pallas import tpu as pltpu
from jax.experimental.pallas import tpu_sc as plsc

_N = 2048
_E = 32768
_HID = 16
_NT = 16
_RPT = _N // _NT
_EPT = _E // _NT
_CHUNK = 128
_NCH = _EPT // _CHUNK


_LW = 128


def _mm1_body(x_ref, w_ref, o_ref):
    r = jnp.dot(x_ref[...], w_ref[...], preferred_element_type=jnp.float32)
    o_ref[...] = jnp.concatenate(
        [r, jnp.zeros((_N, _LW - _HID), jnp.float32)], axis=1)


def _mm2_body(g_ref, w_ref, b_ref, o_ref):
    w = jnp.concatenate(
        [w_ref[...], jnp.zeros((_LW - _HID, _N), jnp.float32)], axis=0)
    o_ref[...] = jnp.dot(g_ref[...], w,
                         preferred_element_type=jnp.float32) + b_ref[...]


_mm1 = pl.pallas_call(
    _mm1_body,
    out_shape=jax.ShapeDtypeStruct((_N, _LW), jnp.float32),
)

_mm2 = pl.pallas_call(
    _mm2_body,
    out_shape=jax.ShapeDtypeStruct((_N, _N), jnp.float32),
)


def _ei_chunk(ei_hbm, o):
    return ei_hbm.at[o // 256, pl.ds(o % 256, _CHUNK)]


def _sc_deg(ei_hbm, dis_hbm, *refs):
    didx = refs[0:_NCH]
    (disbuf, onesbuf, s_deg, sem_i, sem_s) = refs[_NCH:]
    cid = lax.axis_index("c")
    wid = lax.axis_index("s")

    @pl.when(cid == 0)
    def _core0_work():
        rows = pl.ds(wid * _RPT, _RPT)
        ebase = wid * _EPT

        hh = [pltpu.async_copy(
            _ei_chunk(ei_hbm, _E + ebase + j * _CHUNK), didx[j],
            sem_i) for j in range(_NCH)]

        ones = jnp.ones((_HID,), jnp.float32)

        def fill_ones(i, c):
            onesbuf[i, :] = ones
            return c

        lax.fori_loop(0, _RPT, fill_ones, 0)
        pltpu.sync_copy(onesbuf, s_deg.at[rows])
        for h in hh:
            h.wait()
        plsc.subcore_barrier()

        hh = [pltpu.async_copy(onesbuf, s_deg.at[didx[j]], sem_s, add=True)
              for j in range(_NCH)]
        for h in hh:
            h.wait()
        plsc.subcore_barrier()

        pltpu.sync_copy(s_deg.at[rows], disbuf)

        def mk_dis(i, c):
            for i2 in (2 * i, 2 * i + 1):
                d = disbuf[i2, :]
                bits = lax.bitcast_convert_type(d, jnp.int32)
                bits = (jnp.int32(0x5F3759DF)
                        - lax.shift_right_logical(bits, 1))
                y = lax.bitcast_convert_type(bits, jnp.float32)
                half = 0.5 * d
                y = y * (1.5 - half * y * y)
                y = y * (1.5 - half * y * y)
                y = y * (1.5 - half * y * y)
                disbuf[i2, :] = y
            return c

        lax.fori_loop(0, _RPT // 2, mk_dis, 0)
        pltpu.sync_copy(disbuf, dis_hbm.at[rows])


def _sc_body(ei_hbm, hpre_hbm, dis_hbm, b1_hbm, out_hbm, *refs):
    sidx = refs[0:_NCH]
    didx = refs[_NCH:2 * _NCH]
    (rowbig, hbuf, disbuf, accbuf, outbuf, b1buf,
     s_h, s_acc, s_acc2, sem_i, sem_g, sem_s) = refs[2 * _NCH:]
    cid = lax.axis_index("c")
    wid = lax.axis_index("s")

    @pl.when(cid == 0)
    def _core0_work():
        rows = pl.ds(wid * _RPT, _RPT)
        ebase = wid * _EPT

        hh = [pltpu.async_copy(
            _ei_chunk(ei_hbm, ebase + j * _CHUNK), sidx[j], sem_i)
            for j in range(_NCH)]
        hh.append(pltpu.async_copy(hpre_hbm.at[rows], hbuf, sem_i))
        zv = jnp.zeros((_HID,), jnp.float32)

        def fill_zero(i, c):
            for k in range(_LW // _HID):
                outbuf[i, pl.ds(k * _HID, _HID)] = zv
            return c

        lax.fori_loop(0, _RPT, fill_zero, 0)
        for h in hh:
            h.wait()
        hh = [pltpu.async_copy(
            _ei_chunk(ei_hbm, _E + ebase + j * _CHUNK), didx[j],
            sem_i) for j in range(_NCH)]
        hh.append(pltpu.async_copy(dis_hbm.at[rows], disbuf, sem_i))
        hh.append(pltpu.async_copy(b1_hbm, b1buf, sem_i))
        for h in hh:
            h.wait()

        def mk_hs(i, c):
            for i2 in (2 * i, 2 * i + 1):
                accbuf[i2, :] = hbuf[i2, pl.ds(0, _HID)] * disbuf[i2, :]
            return c

        lax.fori_loop(0, _RPT // 2, mk_hs, 0)
        pltpu.sync_copy(accbuf, s_h.at[rows])
        pltpu.sync_copy(accbuf, s_acc.at[rows])
        plsc.subcore_barrier()

        def prop_round(s_to):
            gh = [pltpu.async_copy(
                s_h.at[sidx[j]],
                rowbig.at[pl.ds(j * _CHUNK, _CHUNK)], sem_g)
                for j in range(_NCH)]
            sh = []
            for j in range(_NCH):
                gh[j].wait()
                sh.append(pltpu.async_copy(
                    rowbig.at[pl.ds(j * _CHUNK, _CHUNK)],
                    s_to.at[didx[j]], sem_s, add=True))
            for h in sh:
                h.wait()

        prop_round(s_acc)
        plsc.subcore_barrier()

        pltpu.sync_copy(s_acc.at[rows], accbuf)
        b1v = b1buf[:]

        def mk_h1(i, c):
            for i2 in (2 * i, 2 * i + 1):
                a = accbuf[i2, :] * disbuf[i2, :] + b1v
                a = jnp.maximum(a, 0.0)
                accbuf[i2, :] = a * disbuf[i2, :]
            return c

        lax.fori_loop(0, _RPT // 2, mk_h1, 0)
        pltpu.sync_copy(accbuf, s_h.at[rows])
        pltpu.sync_copy(accbuf, s_acc2.at[rows])
        plsc.subcore_barrier()

        prop_round(s_acc2)
        plsc.subcore_barrier()

        pltpu.sync_copy(s_acc2.at[rows], accbuf)

        def mk_out(i, c):
            for i2 in (2 * i, 2 * i + 1):
                outbuf[i2, pl.ds(0, _HID)] = (
                    accbuf[i2, :] * disbuf[i2, :])
            return c

        lax.fori_loop(0, _RPT // 2, mk_out, 0)
        pltpu.sync_copy(outbuf, out_hbm.at[rows])


_mesh = plsc.VectorSubcoreMesh(core_axis_name="c", subcore_axis_name="s")

_sc_deg_call = functools.partial(
    pl.kernel,
    mesh=_mesh,
    compiler_params=pltpu.CompilerParams(use_tc_tiling_on_sc=False),
    out_type=jax.ShapeDtypeStruct((_N, _HID), jnp.float32),
    scratch_types=(
        [pltpu.VMEM((_CHUNK,), jnp.int32) for _ in range(_NCH)] + [
            pltpu.VMEM((_RPT, _HID), jnp.float32),
            pltpu.VMEM((_RPT, _HID), jnp.float32),
            pltpu.VMEM_SHARED((_N, _HID), jnp.float32),
            pltpu.SemaphoreType.DMA,
            pltpu.SemaphoreType.DMA,
        ]),
)(_sc_deg)

_sc_prop = functools.partial(
    pl.kernel,
    mesh=_mesh,
    compiler_params=pltpu.CompilerParams(use_tc_tiling_on_sc=False),
    out_type=jax.ShapeDtypeStruct((_N, _LW), jnp.float32),
    scratch_types=(
        [pltpu.VMEM((_CHUNK,), jnp.int32) for _ in range(2 * _NCH)] + [
            pltpu.VMEM((_EPT, _HID), jnp.float32),
            pltpu.VMEM((_RPT, _LW), jnp.float32),
            pltpu.VMEM((_RPT, _HID), jnp.float32),
            pltpu.VMEM((_RPT, _HID), jnp.float32),
            pltpu.VMEM((_RPT, _LW), jnp.float32),
            pltpu.VMEM((_HID,), jnp.float32),
            pltpu.VMEM_SHARED((_N, _HID), jnp.float32),
            pltpu.VMEM_SHARED((_N, _HID), jnp.float32),
            pltpu.VMEM_SHARED((_N, _HID), jnp.float32),
            pltpu.SemaphoreType.DMA,
            pltpu.SemaphoreType.DMA,
            pltpu.SemaphoreType.DMA,
        ]),
)(_sc_body)


@jax.jit
def kernel(x, edge_index, W1, b1, W2, b2):
    ei_flat = edge_index.reshape(256, 256)
    dis = _sc_deg_call(ei_flat)
    hpre = _mm1(x, W1)
    g = _sc_prop(ei_flat, hpre, dis, b1)
    out = _mm2(g, W2, b2.reshape(1, _N))
    return out


# speedup vs baseline: 1.1676x; 1.0032x over previous
import functools

import jax
import jax.numpy as jnp
from jax import lax
from jax.experimental import pallas as pl
from jax.experimental.pallas import tpu as pltpu
from jax.experimental.pallas import tpu_sc as plsc

_N = 2048
_E = 32768
_HID = 16
_NT = 16
_RPT = _N // _NT
_EPT = _E // _NT
_CHUNK = 128
_NCH = _EPT // _CHUNK


_LW = 128


def _mm1_body(x_ref, wt_ref, o_ref):
    r = lax.dot_general(x_ref[...], wt_ref[...],
                        (((1,), (1,)), ((), ())),
                        preferred_element_type=jnp.float32)
    o_ref[...] = jnp.concatenate(
        [r, jnp.zeros((_N, _LW - _HID), jnp.float32)], axis=1)


def _mm2_body(g_ref, w_ref, b_ref, o_ref):
    w = jnp.concatenate(
        [w_ref[...], jnp.zeros((_LW - _HID, _N), jnp.float32)], axis=0)
    o_ref[...] = jnp.dot(g_ref[...], w,
                         preferred_element_type=jnp.float32) + b_ref[...]


_mm1 = pl.pallas_call(
    _mm1_body,
    out_shape=jax.ShapeDtypeStruct((_N, _LW), jnp.float32),
)

_mm2 = pl.pallas_call(
    _mm2_body,
    out_shape=jax.ShapeDtypeStruct((_N, _N), jnp.float32),
)


def _ei_chunk(ei_hbm, o):
    return ei_hbm.at[(o % _E) // _CHUNK, o // _E]


def _sc_deg(ei_hbm, dis_hbm, *refs):
    didx = refs[0:_NCH]
    (disbuf, onesbuf, s_deg, sem_i, sem_s) = refs[_NCH:]
    cid = lax.axis_index("c")
    wid = lax.axis_index("s")

    @pl.when(cid == 0)
    def _core0_work():
        rows = pl.ds(wid * _RPT, _RPT)
        ebase = wid * _EPT

        hh = [pltpu.async_copy(
            _ei_chunk(ei_hbm, _E + ebase + j * _CHUNK), didx[j],
            sem_i) for j in range(_NCH)]

        ones = jnp.ones((_HID,), jnp.float32)

        def fill_ones(i, c):
            onesbuf[i, :] = ones
            return c

        lax.fori_loop(0, _RPT, fill_ones, 0)
        pltpu.sync_copy(onesbuf, s_deg.at[rows])
        for h in hh:
            h.wait()
        plsc.subcore_barrier()

        hh = [pltpu.async_copy(onesbuf, s_deg.at[didx[j]], sem_s, add=True)
              for j in range(_NCH)]
        for h in hh:
            h.wait()
        plsc.subcore_barrier()

        pltpu.sync_copy(s_deg.at[rows], disbuf)

        def mk_dis(i, c):
            for i2 in (2 * i, 2 * i + 1):
                d = disbuf[i2, :]
                bits = lax.bitcast_convert_type(d, jnp.int32)
                bits = (jnp.int32(0x5F3759DF)
                        - lax.shift_right_logical(bits, 1))
                y = lax.bitcast_convert_type(bits, jnp.float32)
                half = 0.5 * d
                y = y * (1.5 - half * y * y)
                y = y * (1.5 - half * y * y)
                y = y * (1.5 - half * y * y)
                disbuf[i2, :] = y
            return c

        lax.fori_loop(0, _RPT // 2, mk_dis, 0)
        pltpu.sync_copy(disbuf, dis_hbm.at[rows])


def _sc_body(ei_hbm, hpre_hbm, dis_hbm, b1_hbm, out_hbm, *refs):
    sidx = refs[0:_NCH]
    didx = refs[_NCH:2 * _NCH]
    (rowbig, hbuf, disbuf, accbuf, outbuf, b1buf,
     s_h, s_acc, s_acc2, sem_i, sem_g, sem_s) = refs[2 * _NCH:]
    cid = lax.axis_index("c")
    wid = lax.axis_index("s")

    @pl.when(cid == 0)
    def _core0_work():
        rows = pl.ds(wid * _RPT, _RPT)
        ebase = wid * _EPT

        hh = [pltpu.async_copy(
            _ei_chunk(ei_hbm, ebase + j * _CHUNK), sidx[j], sem_i)
            for j in range(_NCH)]
        hh.append(pltpu.async_copy(hpre_hbm.at[rows], hbuf, sem_i))
        zv = jnp.zeros((_HID,), jnp.float32)

        def fill_zero(i, c):
            for k in range(_LW // _HID):
                outbuf[i, pl.ds(k * _HID, _HID)] = zv
            return c

        lax.fori_loop(0, _RPT, fill_zero, 0)
        for h in hh:
            h.wait()
        hh = [pltpu.async_copy(
            _ei_chunk(ei_hbm, _E + ebase + j * _CHUNK), didx[j],
            sem_i) for j in range(_NCH)]
        hh.append(pltpu.async_copy(dis_hbm.at[rows], disbuf, sem_i))
        hh.append(pltpu.async_copy(b1_hbm, b1buf, sem_i))
        for h in hh:
            h.wait()

        def mk_hs(i, c):
            for i2 in (2 * i, 2 * i + 1):
                accbuf[i2, :] = hbuf[i2, pl.ds(0, _HID)] * disbuf[i2, :]
            return c

        lax.fori_loop(0, _RPT // 2, mk_hs, 0)
        pltpu.sync_copy(accbuf, s_h.at[rows])
        pltpu.sync_copy(accbuf, s_acc.at[rows])
        plsc.subcore_barrier()

        def prop_round(s_to):
            gh = [pltpu.async_copy(
                s_h.at[sidx[j]],
                rowbig.at[pl.ds(j * _CHUNK, _CHUNK)], sem_g)
                for j in range(_NCH)]
            sh = []
            for j in range(_NCH):
                gh[j].wait()
                sh.append(pltpu.async_copy(
                    rowbig.at[pl.ds(j * _CHUNK, _CHUNK)],
                    s_to.at[didx[j]], sem_s, add=True))
            for h in sh:
                h.wait()

        prop_round(s_acc)
        plsc.subcore_barrier()

        pltpu.sync_copy(s_acc.at[rows], accbuf)
        b1v = b1buf[:]

        def mk_h1(i, c):
            for i2 in (2 * i, 2 * i + 1):
                a = accbuf[i2, :] * disbuf[i2, :] + b1v
                a = jnp.maximum(a, 0.0)
                accbuf[i2, :] = a * disbuf[i2, :]
            return c

        lax.fori_loop(0, _RPT // 2, mk_h1, 0)
        pltpu.sync_copy(accbuf, s_h.at[rows])
        pltpu.sync_copy(accbuf, s_acc2.at[rows])
        plsc.subcore_barrier()

        prop_round(s_acc2)
        plsc.subcore_barrier()

        pltpu.sync_copy(s_acc2.at[rows], accbuf)

        def mk_out(i, c):
            for i2 in (2 * i, 2 * i + 1):
                outbuf[i2, pl.ds(0, _HID)] = (
                    accbuf[i2, :] * disbuf[i2, :])
            return c

        lax.fori_loop(0, _RPT // 2, mk_out, 0)
        pltpu.sync_copy(outbuf, out_hbm.at[rows])


_mesh = plsc.VectorSubcoreMesh(core_axis_name="c", subcore_axis_name="s")

_sc_deg_call = functools.partial(
    pl.kernel,
    mesh=_mesh,
    compiler_params=pltpu.CompilerParams(use_tc_tiling_on_sc=False),
    out_type=jax.ShapeDtypeStruct((_N, _HID), jnp.float32),
    scratch_types=(
        [pltpu.VMEM((_CHUNK,), jnp.int32) for _ in range(_NCH)] + [
            pltpu.VMEM((_RPT, _HID), jnp.float32),
            pltpu.VMEM((_RPT, _HID), jnp.float32),
            pltpu.VMEM_SHARED((_N, _HID), jnp.float32),
            pltpu.SemaphoreType.DMA,
            pltpu.SemaphoreType.DMA,
        ]),
)(_sc_deg)

_sc_prop = functools.partial(
    pl.kernel,
    mesh=_mesh,
    compiler_params=pltpu.CompilerParams(use_tc_tiling_on_sc=False),
    out_type=jax.ShapeDtypeStruct((_N, _LW), jnp.float32),
    scratch_types=(
        [pltpu.VMEM((_CHUNK,), jnp.int32) for _ in range(2 * _NCH)] + [
            pltpu.VMEM((_EPT, _HID), jnp.float32),
            pltpu.VMEM((_RPT, _LW), jnp.float32),
            pltpu.VMEM((_RPT, _HID), jnp.float32),
            pltpu.VMEM((_RPT, _HID), jnp.float32),
            pltpu.VMEM((_RPT, _LW), jnp.float32),
            pltpu.VMEM((_HID,), jnp.float32),
            pltpu.VMEM_SHARED((_N, _HID), jnp.float32),
            pltpu.VMEM_SHARED((_N, _HID), jnp.float32),
            pltpu.VMEM_SHARED((_N, _HID), jnp.float32),
            pltpu.SemaphoreType.DMA,
            pltpu.SemaphoreType.DMA,
            pltpu.SemaphoreType.DMA,
        ]),
)(_sc_body)


@jax.jit
def kernel(x, edge_index, W1, b1, W2, b2):
    ei_flat = edge_index.reshape(2, 256, 128).transpose(1, 0, 2)
    dis = _sc_deg_call(ei_flat)
    hpre = _mm1(x, W1.T)
    g = _sc_prop(ei_flat, hpre, dis, b1)
    out = _mm2(g, W2, b2.reshape(1, _N))
    return out
